# Initial kernel scaffold; baseline (speedup 1.0000x reference)
import jax
import jax.numpy as jnp
from jax.experimental import pallas as pl


def kernel(X, edge_index_view0, edge_index_view1, edge_index_view2, W_view0, b_view0, W_view1, b_view1, W_view2, b_view2):
    raise NotImplementedError("write your pallas kernel here")


# speedup vs baseline: 1.9024x; 1.9024x over previous
import functools

import jax
import jax.numpy as jnp
from jax import lax
from jax.experimental import pallas as pl
from jax.experimental.pallas import tpu as pltpu
from jax.experimental.pallas import tpu_sc as plsc

_N = 10000
_NP = 10240
_E = 320000
_D = 128
_NC = 2
_NS = 16
_NW = _NC * _NS
_EPW = _E // _NW
_EPT = _E // _NS
_K = 80
_NB = _EPW // _K
_NBT = _EPT // _K
_RPT = _NP // _NS
_DH = _D // 2

_mesh = plsc.VectorSubcoreMesh(core_axis_name="c", subcore_axis_name="s")
_sc_params = pltpu.CompilerParams(use_tc_tiling_on_sc=False)


@functools.partial(
    pl.kernel,
    out_type=jax.ShapeDtypeStruct((_NC, 6, _NP, 16), jnp.float32),
    mesh=_mesh,
    scratch_types=[
        pltpu.VMEM_SHARED((_NP, 16), jnp.float32),
        pltpu.VMEM_SHARED((_NP, 16), jnp.float32),
        pltpu.VMEM_SHARED((_NP, 16), jnp.float32),
        pltpu.VMEM((_K,), jnp.int32),
        pltpu.VMEM((_K, 16), jnp.float32),
        pltpu.VMEM((_RPT, 16), jnp.float32),
    ],
    compiler_params=_sc_params,
)
def _deg_kernel(i0, i1, i2, i3, i4, i5, zeros_hbm, ones_hbm, out_hbm,
                sp0, sp1, sp2, idx_v, ones_v, zbuf):
    cid = lax.axis_index("c")
    sid = lax.axis_index("s")
    base = (cid * _NS + sid) * _EPW
    rbase = sid * _RPT
    idxs = [i0, i1, i2, i3, i4, i5]
    sps = [sp0, sp1, sp2]
    pltpu.sync_copy(zeros_hbm, zbuf)
    pltpu.sync_copy(ones_hbm, ones_v)
    for g in range(2):
        for j in range(3):
            pltpu.sync_copy(zbuf, sps[j].at[pl.ds(rbase, _RPT)])
        plsc.subcore_barrier()
        for j in range(3):
            def body(b, carry, j=j, g=g):
                pltpu.sync_copy(idxs[g * 3 + j].at[pl.ds(base + b * _K, _K)],
                                idx_v)
                pltpu.sync_copy(ones_v, sps[j].at[idx_v], add=True)
                return carry
            lax.fori_loop(0, _NB, body, 0)
        plsc.subcore_barrier()
        for j in range(3):
            pltpu.sync_copy(sps[j].at[pl.ds(rbase, _RPT)],
                            out_hbm.at[cid, g * 3 + j, pl.ds(rbase, _RPT)])
        plsc.subcore_barrier()


def _matmul3(X, degs_out, Ws):
    blk = 1000

    def body(x_ref, d_ref, w_ref, o_ref):
        x = x_ref[...]
        for v in range(3):
            s = lax.rsqrt(jnp.maximum(d_ref[:, v], 1.0))
            hv = jnp.dot(x * s[:, None], w_ref[v],
                         preferred_element_type=jnp.float32)
            o_ref[0, v] = hv[:, :_DH]
            o_ref[1, v] = hv[:, _DH:]

    return pl.pallas_call(
        body,
        grid=(_N // blk,),
        in_specs=[
            pl.BlockSpec((blk, _D), lambda i: (i, 0)),
            pl.BlockSpec((blk, 3), lambda i: (i, 0)),
            pl.BlockSpec((3, _D, _D), lambda i: (0, 0, 0)),
        ],
        out_specs=pl.BlockSpec((2, 3, blk, _DH), lambda i: (0, 0, i, 0)),
        out_shape=jax.ShapeDtypeStruct((2, 3, _N, _DH), jnp.float32),
    )(X, degs_out, Ws)


@functools.partial(
    pl.kernel,
    out_type=jax.ShapeDtypeStruct((3, _NC, _NP, _DH), jnp.float32),
    mesh=_mesh,
    scratch_types=[
        pltpu.VMEM_SHARED((_NP, _DH), jnp.float32),
        pltpu.VMEM((_K,), jnp.int32),
        pltpu.VMEM((_K,), jnp.int32),
        pltpu.VMEM((_K, _DH), jnp.float32),
        pltpu.VMEM((_RPT, _DH), jnp.float32),
        pltpu.SemaphoreType.DMA,
    ],
    compiler_params=_sc_params,
)
def _scatter_kernel(h2_hbm, s0, s1, s2, d0, d1, d2, zeros_hbm, out_hbm,
                    agg, idx_s, idx_d, rows, zbuf, sem):
    cid = lax.axis_index("c")
    sid = lax.axis_index("s")
    base = cid * _E + sid * _EPT
    rbase = sid * _RPT
    srcs = [s0, s1, s2]
    dsts = [d0, d1, d2]
    pltpu.sync_copy(zeros_hbm, zbuf)
    for v in range(3):
        pltpu.sync_copy(zbuf, agg.at[pl.ds(rbase, _RPT)])
        plsc.subcore_barrier()

        def body(b, carry, v=v):
            off = base + b * _K
            doff = sid * _EPT + b * _K
            pltpu.sync_copy(srcs[v].at[pl.ds(off, _K)], idx_s)
            pltpu.sync_copy(dsts[v].at[pl.ds(doff, _K)], idx_d)
            pltpu.async_copy(h2_hbm.at[idx_s], rows, sem).wait()
            pltpu.sync_copy(rows, agg.at[idx_d], add=True)
            return carry

        lax.fori_loop(0, _NBT, body, 0)
        plsc.subcore_barrier()
        pltpu.sync_copy(agg.at[pl.ds(rbase, _RPT)],
                        out_hbm.at[v, cid, pl.ds(rbase, _RPT)])
        plsc.subcore_barrier()


def _finalize(partials, degs_in, bs):
    blk = 1000

    def body(p_ref, d_ref, b_ref, o_ref):
        acc = jnp.zeros((blk, _D), jnp.float32)
        for v in range(3):
            r = lax.rsqrt(jnp.maximum(d_ref[:, v], 1.0))
            full = jnp.concatenate([p_ref[v, 0], p_ref[v, 1]], axis=1)
            acc += full * r[:, None] + b_ref[v][None, :]
        o_ref[...] = acc * (1.0 / 3.0)

    return pl.pallas_call(
        body,
        grid=(_N // blk,),
        in_specs=[
            pl.BlockSpec((3, _NC, blk, _DH), lambda i: (0, 0, i, 0)),
            pl.BlockSpec((blk, 3), lambda i: (i, 0)),
            pl.BlockSpec((3, _D), lambda i: (0, 0)),
        ],
        out_specs=pl.BlockSpec((blk, _D), lambda i: (i, 0)),
        out_shape=jax.ShapeDtypeStruct((_N, _D), jnp.float32),
    )(partials, degs_in, bs)


def kernel(X, edge_index_view0, edge_index_view1, edge_index_view2,
           W_view0, b_view0, W_view1, b_view1, W_view2, b_view2):
    eis = [edge_index_view0, edge_index_view1, edge_index_view2]
    zeros16 = jnp.zeros((_RPT, 16), jnp.float32)
    ones16 = jnp.ones((_K, 16), jnp.float32)
    deg_part = _deg_kernel(eis[0][0], eis[0][1], eis[1][0], eis[1][1],
                           eis[2][0], eis[2][1], zeros16, ones16)
    degs = deg_part[0, :, :_N, 0] + deg_part[1, :, :_N, 0]
    degs_out = degs[0::2].T
    degs_in = degs[1::2].T

    Ws = jnp.stack([W_view0, W_view1, W_view2])
    bs = jnp.stack([b_view0, b_view1, b_view2])
    h = _matmul3(X, degs_out, Ws)
    h2 = h.reshape(2 * 3 * _N, _DH)

    zeros64 = jnp.zeros((_RPT, _DH), jnp.float32)
    srcs2 = [jnp.concatenate([eis[v][0] + v * _N,
                              eis[v][0] + (3 * _N + v * _N)])
             for v in range(3)]
    parts = _scatter_kernel(
        h2,
        srcs2[0], srcs2[1], srcs2[2],
        eis[0][1], eis[1][1], eis[2][1],
        zeros64)
    parts = parts[:, :, :_N, :]

    return _finalize(parts, degs_in, bs)


# speedup vs baseline: 4.7315x; 2.4871x over previous
import functools

import jax
import jax.numpy as jnp
from jax import lax
from jax.experimental import pallas as pl
from jax.experimental.pallas import tpu as pltpu
from jax.experimental.pallas import tpu_sc as plsc

_N = 10000
_NP = 10240
_E = 320000
_D = 128
_NC = 2
_NS = 16
_NW = _NC * _NS
_EPW = _E // _NW
_EPT = _E // _NS
_K = 80
_NB = _EPW // _K
_NBT = _EPT // _K
_RPT = _NP // _NS
_DH = _D // 2

_mesh = plsc.VectorSubcoreMesh(core_axis_name="c", subcore_axis_name="s")
_sc_params = pltpu.CompilerParams(use_tc_tiling_on_sc=False)


@functools.partial(
    pl.kernel,
    out_type=jax.ShapeDtypeStruct((_NC, 6, _NP, 16), jnp.float32),
    mesh=_mesh,
    scratch_types=[
        pltpu.VMEM_SHARED((_NP, 16), jnp.float32),
        pltpu.VMEM_SHARED((_NP, 16), jnp.float32),
        pltpu.VMEM((_NB, _K), jnp.int32),
        pltpu.VMEM((_K, 16), jnp.float32),
        pltpu.SemaphoreType.DMA,
    ],
    compiler_params=_sc_params,
)
def _deg_kernel(i0, i1, i2, i3, i4, i5, zeros_hbm, ones_hbm, out_hbm,
                sp0, sp1, idx2d, ones_v, sem):
    cid = lax.axis_index("c")
    sid = lax.axis_index("s")
    brow = (cid * _NS + sid) * _NB
    rbase = sid * _RPT
    idxs = [i0, i1, i2, i3, i4, i5]
    sps = [sp0, sp1]
    pltpu.sync_copy(ones_hbm, ones_v)
    for g in range(3):
        for j in range(2):
            pltpu.sync_copy(zeros_hbm, sps[j].at[pl.ds(rbase, _RPT)])
        plsc.subcore_barrier()
        for j in range(2):
            pltpu.sync_copy(idxs[g * 2 + j].at[pl.ds(brow, _NB)], idx2d)

            def fire(b, carry, j=j):
                pltpu.async_copy(ones_v, sps[j].at[idx2d.at[b]], sem,
                                 add=True)
                return carry
            lax.fori_loop(0, _NB, fire, 0)

            def drain(b, carry):
                pltpu.make_async_copy(ones_hbm, ones_v, sem).wait()
                return carry
            lax.fori_loop(0, _NB, drain, 0)
        plsc.subcore_barrier()
        for j in range(2):
            pltpu.sync_copy(sps[j].at[pl.ds(rbase, _RPT)],
                            out_hbm.at[cid, g * 2 + j, pl.ds(rbase, _RPT)])
        plsc.subcore_barrier()


def _matmul3(X, degs_out, Ws):
    blk = 1000

    def body(x_ref, d_ref, w_ref, o_ref):
        x = x_ref[...]
        for v in range(3):
            s = lax.rsqrt(jnp.maximum(d_ref[:, v], 1.0))
            hv = jnp.dot(x * s[:, None], w_ref[v],
                         preferred_element_type=jnp.float32)
            o_ref[0, v] = hv[:, :_DH]
            o_ref[1, v] = hv[:, _DH:]

    return pl.pallas_call(
        body,
        grid=(_N // blk,),
        in_specs=[
            pl.BlockSpec((blk, _D), lambda i: (i, 0)),
            pl.BlockSpec((blk, 3), lambda i: (i, 0)),
            pl.BlockSpec((3, _D, _D), lambda i: (0, 0, 0)),
        ],
        out_specs=pl.BlockSpec((2, 3, blk, _DH), lambda i: (0, 0, i, 0)),
        out_shape=jax.ShapeDtypeStruct((2, 3, _N, _DH), jnp.float32),
    )(X, degs_out, Ws)


_NBUF = 5
_NBH = _NBT // 2
_NG = _NBH // _NBUF


@functools.partial(
    pl.kernel,
    out_type=jax.ShapeDtypeStruct((3, _NC, _NP, _DH), jnp.float32),
    mesh=_mesh,
    scratch_types=[
        pltpu.VMEM_SHARED((_NP, _DH), jnp.float32),
        pltpu.VMEM((_NBH, _K), jnp.int32),
        pltpu.VMEM((_NBH, _K), jnp.int32),
        [pltpu.VMEM((_K, _DH), jnp.float32) for _ in range(_NBUF)],
        pltpu.SemaphoreType.DMA,
        pltpu.SemaphoreType.DMA,
    ],
    compiler_params=_sc_params,
)
def _scatter_kernel(h2_hbm, s0, s1, s2, d0, d1, d2, zeros_hbm, out_hbm,
                    agg, src2d, dst2d, rows, sem_g, sem_s):
    cid = lax.axis_index("c")
    sid = lax.axis_index("s")
    brow = cid * (_E // _K) + sid * _NBT
    drow = sid * _NBT
    rbase = sid * _RPT
    srcs = [s0, s1, s2]
    dsts = [d0, d1, d2]
    for v in range(3):
        pltpu.sync_copy(zeros_hbm, agg.at[pl.ds(rbase, _RPT)])
        plsc.subcore_barrier()
        for hh in range(2):
            pltpu.sync_copy(srcs[v].at[pl.ds(brow + hh * _NBH, _NBH)], src2d)
            pltpu.sync_copy(dsts[v].at[pl.ds(drow + hh * _NBH, _NBH)], dst2d)

            for j in range(_NBUF):
                pltpu.async_copy(h2_hbm.at[src2d.at[j]], rows[j], sem_g)

            def group(g, carry):
                b0 = g * _NBUF
                for j in range(_NBUF):
                    pltpu.make_async_copy(zeros_hbm.at[pl.ds(0, _K)],
                                          rows[j], sem_g).wait()
                for j in range(_NBUF):
                    pltpu.async_copy(rows[j], agg.at[dst2d.at[b0 + j]],
                                     sem_s, add=True)
                for j in range(_NBUF):
                    pltpu.make_async_copy(zeros_hbm.at[pl.ds(0, _K)],
                                          rows[j], sem_s).wait()

                @pl.when(g < _NG - 1)
                def _():
                    for j in range(_NBUF):
                        pltpu.async_copy(h2_hbm.at[src2d.at[b0 + _NBUF + j]],
                                         rows[j], sem_g)
                return carry

            lax.fori_loop(0, _NG, group, 0)
        plsc.subcore_barrier()
        pltpu.sync_copy(agg.at[pl.ds(rbase, _RPT)],
                        out_hbm.at[v, cid, pl.ds(rbase, _RPT)])
        plsc.subcore_barrier()


def _finalize(partials, degs_in, bs):
    blk = 1000

    def body(p_ref, d_ref, b_ref, o_ref):
        acc = jnp.zeros((blk, _D), jnp.float32)
        for v in range(3):
            r = lax.rsqrt(jnp.maximum(d_ref[:, v], 1.0))
            full = jnp.concatenate([p_ref[v, 0], p_ref[v, 1]], axis=1)
            acc += full * r[:, None] + b_ref[v][None, :]
        o_ref[...] = acc * (1.0 / 3.0)

    return pl.pallas_call(
        body,
        grid=(_N // blk,),
        in_specs=[
            pl.BlockSpec((3, _NC, blk, _DH), lambda i: (0, 0, i, 0)),
            pl.BlockSpec((blk, 3), lambda i: (i, 0)),
            pl.BlockSpec((3, _D), lambda i: (0, 0)),
        ],
        out_specs=pl.BlockSpec((blk, _D), lambda i: (i, 0)),
        out_shape=jax.ShapeDtypeStruct((_N, _D), jnp.float32),
    )(partials, degs_in, bs)


def kernel(X, edge_index_view0, edge_index_view1, edge_index_view2,
           W_view0, b_view0, W_view1, b_view1, W_view2, b_view2):
    eis = [edge_index_view0, edge_index_view1, edge_index_view2]
    zeros16 = jnp.zeros((_RPT, 16), jnp.float32)
    ones16 = jnp.ones((_K, 16), jnp.float32)
    deg_part = _deg_kernel(
        eis[0][0].reshape(-1, _K), eis[0][1].reshape(-1, _K),
        eis[1][0].reshape(-1, _K), eis[1][1].reshape(-1, _K),
        eis[2][0].reshape(-1, _K), eis[2][1].reshape(-1, _K),
        zeros16, ones16)
    degs = deg_part[0, :, :_N, 0] + deg_part[1, :, :_N, 0]
    degs_out = degs[0::2].T
    degs_in = degs[1::2].T

    Ws = jnp.stack([W_view0, W_view1, W_view2])
    bs = jnp.stack([b_view0, b_view1, b_view2])
    h = _matmul3(X, degs_out, Ws)
    h2 = h.reshape(2 * 3 * _N, _DH)

    zeros64 = jnp.zeros((_RPT, _DH), jnp.float32)
    srcs2 = [jnp.concatenate([eis[v][0] + v * _N,
                              eis[v][0] + (3 * _N + v * _N)]).reshape(-1, _K)
             for v in range(3)]
    parts = _scatter_kernel(
        h2,
        srcs2[0], srcs2[1], srcs2[2],
        eis[0][1].reshape(-1, _K), eis[1][1].reshape(-1, _K),
        eis[2][1].reshape(-1, _K),
        zeros64)
    parts = parts[:, :, :_N, :]

    return _finalize(parts, degs_in, bs)


# speedup vs baseline: 7.3231x; 1.5477x over previous
import functools

import jax
import jax.numpy as jnp
from jax import lax
from jax.experimental import pallas as pl
from jax.experimental.pallas import tpu as pltpu
from jax.experimental.pallas import tpu_sc as plsc

_N = 10000
_NP = 10240
_E = 320000
_D = 128
_NC = 2
_NS = 16
_NW = _NC * _NS
_EPW = _E // _NW
_EPT = _E // _NS
_K = 80
_NB = _EPW // _K
_NBT = _EPT // _K
_RPT = _NP // _NS
_DH = _D // 2

_mesh = plsc.VectorSubcoreMesh(core_axis_name="c", subcore_axis_name="s")
_sc_params = pltpu.CompilerParams(use_tc_tiling_on_sc=False)


@functools.partial(
    pl.kernel,
    out_type=jax.ShapeDtypeStruct((_NC, 6, _NP, 16), jnp.float32),
    mesh=_mesh,
    scratch_types=[
        pltpu.VMEM_SHARED((_NP, 16), jnp.float32),
        pltpu.VMEM_SHARED((_NP, 16), jnp.float32),
        pltpu.VMEM((_NB, _K), jnp.int32),
        pltpu.VMEM((_K, 16), jnp.float32),
        pltpu.SemaphoreType.DMA,
    ],
    compiler_params=_sc_params,
)
def _deg_kernel(i0, i1, i2, i3, i4, i5, zeros_hbm, ones_hbm, out_hbm,
                sp0, sp1, idx2d, ones_v, sem):
    cid = lax.axis_index("c")
    sid = lax.axis_index("s")
    brow = (cid * _NS + sid) * _NB
    rbase = sid * _RPT
    idxs = [i0, i1, i2, i3, i4, i5]
    sps = [sp0, sp1]
    pltpu.sync_copy(ones_hbm, ones_v)
    for g in range(3):
        for j in range(2):
            pltpu.sync_copy(zeros_hbm, sps[j].at[pl.ds(rbase, _RPT)])
        plsc.subcore_barrier()
        for j in range(2):
            pltpu.sync_copy(idxs[g * 2 + j].at[pl.ds(brow, _NB)], idx2d)

            def fire(b, carry, j=j):
                pltpu.async_copy(ones_v, sps[j].at[idx2d.at[b]], sem,
                                 add=True)
                return carry
            lax.fori_loop(0, _NB, fire, 0)

            def drain(b, carry):
                pltpu.make_async_copy(ones_hbm, ones_v, sem).wait()
                return carry
            lax.fori_loop(0, _NB, drain, 0)
        plsc.subcore_barrier()
        for j in range(2):
            pltpu.sync_copy(sps[j].at[pl.ds(rbase, _RPT)],
                            out_hbm.at[cid, g * 2 + j, pl.ds(rbase, _RPT)])
        plsc.subcore_barrier()


def _matmul3(X, deg_part, Ws):
    blk = 1000

    def body(x_ref, d_ref, w_ref, o_ref):
        x = x_ref[...]
        for v in range(3):
            deg = d_ref[0, v, :, 0] + d_ref[1, v, :, 0]
            s = lax.rsqrt(jnp.maximum(deg, 1.0))
            hv = jnp.dot(x * s[:, None], w_ref[v],
                         preferred_element_type=jnp.float32)
            o_ref[0, v] = hv[:, :_DH]
            o_ref[1, v] = hv[:, _DH:]

    return pl.pallas_call(
        body,
        grid=(_N // blk,),
        in_specs=[
            pl.BlockSpec((blk, _D), lambda i: (i, 0)),
            pl.BlockSpec((2, 3, blk, 16), lambda i: (0, 0, i, 0)),
            pl.BlockSpec((3, _D, _D), lambda i: (0, 0, 0)),
        ],
        out_specs=pl.BlockSpec((2, 3, blk, _DH), lambda i: (0, 0, i, 0)),
        out_shape=jax.ShapeDtypeStruct((2, 3, _N, _DH), jnp.float32),
    )(X, deg_part, Ws)


_NBUF = 5
_NBH = _NBT // 2
_NG = _NBH // _NBUF
_NPAIR = (_NG - 1) // 2


@functools.partial(
    pl.kernel,
    out_type=jax.ShapeDtypeStruct((3, _NC, _NP, _DH), jnp.float32),
    mesh=_mesh,
    scratch_types=[
        pltpu.VMEM_SHARED((_NP, _DH), jnp.float32),
        pltpu.VMEM((_NBH, _K), jnp.int32),
        pltpu.VMEM((_NBH, _K), jnp.int32),
        [pltpu.VMEM((_K, _DH), jnp.float32) for _ in range(2 * _NBUF)],
        [pltpu.SemaphoreType.DMA for _ in range(4)],
    ],
    compiler_params=_sc_params,
)
def _scatter_kernel(h2_hbm, s0, s1, s2, d0, d1, d2, zeros_hbm, out_hbm,
                    agg, src2d, dst2d, rows, sems):
    cid = lax.axis_index("c")
    sid = lax.axis_index("s")
    brow = cid * (_E // _K) + sid * _NBT
    drow = sid * _NBT
    rbase = sid * _RPT
    srcs = [s0, s1, s2]
    dsts = [d0, d1, d2]
    sem_g = [sems[0], sems[1]]
    sem_s = [sems[2], sems[3]]

    def fire_g(g0, bank):
        for j in range(_NBUF):
            pltpu.async_copy(h2_hbm.at[src2d.at[g0 * _NBUF + j]],
                             rows[bank * _NBUF + j], sem_g[bank])

    def fire_s(g0, bank):
        for j in range(_NBUF):
            pltpu.async_copy(rows[bank * _NBUF + j],
                             agg.at[dst2d.at[g0 * _NBUF + j]],
                             sem_s[bank], add=True)

    def drain(sem):
        for j in range(_NBUF):
            pltpu.make_async_copy(zeros_hbm.at[pl.ds(0, _K)], rows[0],
                                  sem).wait()

    for v in range(3):
        pltpu.sync_copy(zeros_hbm, agg.at[pl.ds(rbase, _RPT)])
        plsc.subcore_barrier()
        for hh in range(2):
            pltpu.sync_copy(srcs[v].at[pl.ds(brow + hh * _NBH, _NBH)], src2d)
            pltpu.sync_copy(dsts[v].at[pl.ds(drow + hh * _NBH, _NBH)], dst2d)

            fire_g(0, 0)
            fire_g(1, 1)

            def pair(t, carry):
                g0 = 2 * t
                drain(sem_g[0])
                fire_s(g0, 0)
                drain(sem_g[1])
                drain(sem_s[0])
                fire_g(g0 + 2, 0)
                fire_s(g0 + 1, 1)
                drain(sem_s[1])

                @pl.when(t < _NPAIR - 1)
                def _():
                    fire_g(g0 + 3, 1)
                return carry

            lax.fori_loop(0, _NPAIR, pair, 0)
            drain(sem_g[0])
            fire_s(_NG - 1, 0)
            drain(sem_s[0])
        plsc.subcore_barrier()
        pltpu.sync_copy(agg.at[pl.ds(rbase, _RPT)],
                        out_hbm.at[v, cid, pl.ds(rbase, _RPT)])
        plsc.subcore_barrier()


def _finalize(partials, deg_part, bs):
    blk = 1000

    def body(p_ref, d_ref, b_ref, o_ref):
        acc = jnp.zeros((blk, _D), jnp.float32)
        for v in range(3):
            deg = d_ref[0, v, :, 0] + d_ref[1, v, :, 0]
            r = lax.rsqrt(jnp.maximum(deg, 1.0))
            full = jnp.concatenate([p_ref[v, 0], p_ref[v, 1]], axis=1)
            acc += full * r[:, None] + b_ref[v][None, :]
        o_ref[...] = acc * (1.0 / 3.0)

    return pl.pallas_call(
        body,
        grid=(_N // blk,),
        in_specs=[
            pl.BlockSpec((3, _NC, blk, _DH), lambda i: (0, 0, i, 0)),
            pl.BlockSpec((2, 3, blk, 16), lambda i: (0, 1, i, 0)),
            pl.BlockSpec((3, _D), lambda i: (0, 0)),
        ],
        out_specs=pl.BlockSpec((blk, _D), lambda i: (i, 0)),
        out_shape=jax.ShapeDtypeStruct((_N, _D), jnp.float32),
    )(partials, deg_part, bs)


def kernel(X, edge_index_view0, edge_index_view1, edge_index_view2,
           W_view0, b_view0, W_view1, b_view1, W_view2, b_view2):
    eis = [edge_index_view0, edge_index_view1, edge_index_view2]
    zeros16 = jnp.zeros((_RPT, 16), jnp.float32)
    ones16 = jnp.ones((_K, 16), jnp.float32)
    deg_part = _deg_kernel(
        eis[0][0].reshape(-1, _K), eis[1][0].reshape(-1, _K),
        eis[2][0].reshape(-1, _K), eis[0][1].reshape(-1, _K),
        eis[1][1].reshape(-1, _K), eis[2][1].reshape(-1, _K),
        zeros16, ones16)

    Ws = jnp.stack([W_view0, W_view1, W_view2])
    bs = jnp.stack([b_view0, b_view1, b_view2])
    h = _matmul3(X, deg_part, Ws)
    h2 = h.reshape(2 * 3 * _N, _DH)

    zeros64 = jnp.zeros((_RPT, _DH), jnp.float32)
    srcs2 = [jnp.concatenate([eis[v][0] + v * _N,
                              eis[v][0] + (3 * _N + v * _N)]).reshape(-1, _K)
             for v in range(3)]
    parts = _scatter_kernel(
        h2,
        srcs2[0], srcs2[1], srcs2[2],
        eis[0][1].reshape(-1, _K), eis[1][1].reshape(-1, _K),
        eis[2][1].reshape(-1, _K),
        zeros64)

    return _finalize(parts, deg_part, bs)
